# baseline (speedup 1.0000x reference)
import functools

import jax
import jax.numpy as jnp
from jax import lax
from jax.experimental import pallas as pl
from jax.experimental.pallas import tpu as pltpu
from jax.experimental.pallas import tpu_sc as plsc

_INFO = plsc.get_sparse_core_info()
_NC, _NS = _INFO.num_cores, _INFO.num_subcores
_NW = _NC * _NS

_NBUF = 4
_LAG = 2


@functools.partial(jax.jit, static_argnums=(2, 3, 4, 5))
def _sc_gather(table, idx, b_per_w, chunk, n_chunks, seq=200):
    D = table.shape[1]
    B = idx.shape[0]
    rows_per_chunk = chunk // seq
    mesh = plsc.VectorSubcoreMesh(core_axis_name="c", subcore_axis_name="s")

    scratch = (
        [pltpu.VMEM((chunk,), jnp.int32) for _ in range(_NBUF)]
        + [pltpu.VMEM((chunk, D), table.dtype) for _ in range(_NBUF)]
        + [pltpu.SemaphoreType.DMA for _ in range(2 * _NBUF)]
    )

    @functools.partial(
        pl.kernel,
        mesh=mesh,
        out_type=jax.ShapeDtypeStruct((B // seq, seq, D), table.dtype),
        scratch_types=scratch,
        compiler_params=pltpu.CompilerParams(use_tc_tiling_on_sc=False),
    )
    def k(table_hbm, idx_hbm, out_hbm, *bufs):
        idxs = bufs[:_NBUF]
        rows = bufs[_NBUF : 2 * _NBUF]
        gsem = bufs[2 * _NBUF : 3 * _NBUF]
        ssem = bufs[3 * _NBUF :]

        wid = lax.axis_index("s") * _NC + lax.axis_index("c")
        w_base = wid * b_per_w

        def fill(g, b):
            base = w_base + g * chunk
            pltpu.sync_copy(idx_hbm.at[pl.ds(base, chunk)], idxs[b])
            pltpu.async_copy(table_hbm.at[idxs[b]], rows[b], gsem[b])

        def drain(g, b):
            pltpu.make_async_copy(table_hbm.at[idxs[b]], rows[b], gsem[b]).wait()
            brow = (w_base + g * chunk) // seq
            for r in range(rows_per_chunk):
                pltpu.async_copy(
                    rows[b].at[pl.ds(r * seq, seq)],
                    out_hbm.at[brow + r],
                    ssem[b],
                )

        def store_wait(g, b):
            for r in range(rows_per_chunk):
                pltpu.make_async_copy(
                    rows[b].at[pl.ds(r * seq, seq)], out_hbm.at[0], ssem[b]
                ).wait()

        for g in range(_NBUF):
            if g >= _LAG:
                drain(g - _LAG, g - _LAG)
            fill(g, g)

        def round_body(r, carry):
            for b in range(_NBUF):
                g = r * _NBUF + b
                drain(g - _LAG, (b + _NBUF - _LAG) % _NBUF)
                store_wait(g - _NBUF, b)
                fill(g, b)
            return carry

        lax.fori_loop(1, n_chunks // _NBUF, round_body, 0)

        for i in range(_LAG):
            g = n_chunks - _LAG + i
            drain(g, g % _NBUF)
        for i in range(_NBUF):
            g = n_chunks - _NBUF + i
            store_wait(g, g % _NBUF)

    return k(table, idx)


def kernel(x, table):
    Bt, S = x.shape
    D = table.shape[1]
    B = Bt * S
    idx = x.reshape(B).astype(jnp.int32)
    chunk = 400
    assert B % (_NW * chunk) == 0
    b_per_w = B // _NW
    n_chunks = b_per_w // chunk
    assert n_chunks % _NBUF == 0
    out = _sc_gather(table, idx, b_per_w, chunk, n_chunks, S)
    return out
